# baseline (speedup 1.0000x reference)
import jax
import jax.numpy as jnp
from jax import lax
from jax.experimental import pallas as pl
from jax.experimental.pallas import tpu as pltpu
from jax.experimental.pallas import tpu_sc as plsc

_B = 16384
_D = 64
_NC = 2
_NS = 16
_NW = _NC * _NS
_BPW = _B // _NW
_H = _BPW // 2
_L = 16

_N = 1000000
_NL = _N // 2


def _svd_score_body(uid_hbm, iid_hbm, ut_hbm, it_hbm, ub_hbm, ib_hbm, gb_hbm,
                    out_hbm, idx_u, idx_i, idxp_u, idxp_i, u2, v2, bu_v, bi_v,
                    gb_v, out_v, sem, bsem):
    wid = lax.axis_index("s") * _NC + lax.axis_index("c")
    base = wid * _BPW

    pltpu.sync_copy(uid_hbm.at[pl.ds(base, _BPW)], idx_u)
    pltpu.sync_copy(iid_hbm.at[pl.ds(base, _BPW)], idx_i)

    cp_bu = pltpu.async_copy(ub_hbm.at[idx_u], bu_v, bsem)
    cp_bi = pltpu.async_copy(ib_hbm.at[idx_i], bi_v, bsem)
    pltpu.sync_copy(gb_hbm, gb_v)

    @pl.loop(0, _BPW, step=_L)
    def _(r0):
        idxp_u[pl.ds(r0, _L)] = lax.shift_right_logical(idx_u[pl.ds(r0, _L)], 1)
        idxp_i[pl.ds(r0, _L)] = lax.shift_right_logical(idx_i[pl.ds(r0, _L)], 1)

    cp_bu.wait()
    cp_bi.wait()

    gb_vec = gb_v[...]
    lane = lax.iota(jnp.int32, _L)
    onehots = [(lane == l).astype(jnp.float32) for l in range(_L)]

    for half in range(2):
        hbase = half * _H
        cp_u = pltpu.async_copy(ut_hbm.at[idxp_u.at[pl.ds(hbase, _H)]], u2, sem)
        cp_v = pltpu.async_copy(it_hbm.at[idxp_i.at[pl.ds(hbase, _H)]], v2, sem)
        cp_u.wait()
        cp_v.wait()

        @pl.loop(0, _H, step=_L)
        def _(r0):
            res = (bu_v[pl.ds(hbase + r0, _L)] + bi_v[pl.ds(hbase + r0, _L)]
                   + gb_vec)
            iu_vec = idx_u[pl.ds(hbase + r0, _L)]
            iv_vec = idx_i[pl.ds(hbase + r0, _L)]
            for l in range(_L):
                r = r0 + l
                su = (iu_vec[l] & 1) * _D
                sv = (iv_vec[l] & 1) * _D
                acc = u2[r, pl.ds(su, _L)] * v2[r, pl.ds(sv, _L)]
                for c in range(1, _D // _L):
                    acc += (u2[r, pl.ds(su + c * _L, _L)]
                            * v2[r, pl.ds(sv + c * _L, _L)])
                res += jnp.sum(acc) * onehots[l]
            out_v[pl.ds(hbase + r0, _L)] = res

    pltpu.sync_copy(out_v, out_hbm.at[pl.ds(base, _BPW)])


@jax.jit
def kernel(user_ids, item_ids, user_table, item_table, user_bias, item_bias,
           global_bias):
    mesh = plsc.VectorSubcoreMesh(core_axis_name="c", subcore_axis_name="s")

    score = pl.kernel(
        _svd_score_body,
        out_type=jax.ShapeDtypeStruct((_B,), jnp.float32),
        mesh=mesh,
        compiler_params=pltpu.CompilerParams(use_tc_tiling_on_sc=True,
                                             needs_layout_passes=False),
        scratch_types=[
            pltpu.VMEM((_BPW,), jnp.int32),
            pltpu.VMEM((_BPW,), jnp.int32),
            pltpu.VMEM((_BPW,), jnp.int32),
            pltpu.VMEM((_BPW,), jnp.int32),
            pltpu.VMEM((_H, 2 * _D), jnp.float32),
            pltpu.VMEM((_H, 2 * _D), jnp.float32),
            pltpu.VMEM((_BPW,), jnp.float32),
            pltpu.VMEM((_BPW,), jnp.float32),
            pltpu.VMEM((_L,), jnp.float32),
            pltpu.VMEM((_BPW,), jnp.float32),
            pltpu.SemaphoreType.DMA,
            pltpu.SemaphoreType.DMA,
        ],
    )

    ut2 = user_table.reshape(_NL, 2 * _D)
    it2 = item_table.reshape(_NL, 2 * _D)
    gb_b = jnp.broadcast_to(global_bias, (_L,))
    return score(user_ids.astype(jnp.int32), item_ids.astype(jnp.int32),
                 ut2, it2, user_bias, item_bias, gb_b)


# speedup vs baseline: 1.0698x; 1.0698x over previous
import jax
import jax.numpy as jnp
from jax import lax
from jax.experimental import pallas as pl
from jax.experimental.pallas import tpu as pltpu
from jax.experimental.pallas import tpu_sc as plsc

_B = 16384
_D = 64
_NC = 2
_NS = 16
_NW = _NC * _NS
_BPW = _B // _NW
_H = _BPW // 2
_L = 16

_N = 1000000


def _svd_score_body(uid_hbm, iid_hbm, ut_hbm, it_hbm, ub_hbm, ib_hbm, gb_hbm,
                    out_hbm, idx_u, idx_i, u2, v2, bu_v, bi_v,
                    gb_v, out_v, sem, bsem):
    wid = lax.axis_index("s") * _NC + lax.axis_index("c")
    base = wid * _BPW

    pltpu.sync_copy(uid_hbm.at[pl.ds(base, _BPW)], idx_u)
    pltpu.sync_copy(iid_hbm.at[pl.ds(base, _BPW)], idx_i)

    cp_bu = pltpu.async_copy(ub_hbm.at[idx_u], bu_v, bsem)
    cp_bi = pltpu.async_copy(ib_hbm.at[idx_i], bi_v, bsem)
    pltpu.sync_copy(gb_hbm, gb_v)

    cp_bu.wait()
    cp_bi.wait()

    gb_vec = gb_v[...]
    lane = lax.iota(jnp.int32, _L)
    onehots = [(lane == l).astype(jnp.float32) for l in range(_L)]

    for half in range(2):
        hbase = half * _H
        cp_u = pltpu.async_copy(ut_hbm.at[idx_u.at[pl.ds(hbase, _H)]], u2, sem)
        cp_v = pltpu.async_copy(it_hbm.at[idx_i.at[pl.ds(hbase, _H)]], v2, sem)
        cp_u.wait()
        cp_v.wait()

        @pl.loop(0, _H, step=_L)
        def _(r0):
            res = (bu_v[pl.ds(hbase + r0, _L)] + bi_v[pl.ds(hbase + r0, _L)]
                   + gb_vec)
            for l in range(_L):
                r = r0 + l
                acc = u2[r, pl.ds(0, _L)] * v2[r, pl.ds(0, _L)]
                for c in range(1, _D // _L):
                    acc += (u2[r, pl.ds(c * _L, _L)]
                            * v2[r, pl.ds(c * _L, _L)])
                res += jnp.sum(acc) * onehots[l]
            out_v[pl.ds(hbase + r0, _L)] = res

    pltpu.sync_copy(out_v, out_hbm.at[pl.ds(base, _BPW)])


@jax.jit
def kernel(user_ids, item_ids, user_table, item_table, user_bias, item_bias,
           global_bias):
    mesh = plsc.VectorSubcoreMesh(core_axis_name="c", subcore_axis_name="s")

    score = pl.kernel(
        _svd_score_body,
        out_type=jax.ShapeDtypeStruct((_B,), jnp.float32),
        mesh=mesh,
        compiler_params=pltpu.CompilerParams(use_tc_tiling_on_sc=False,
                                             needs_layout_passes=False),
        scratch_types=[
            pltpu.VMEM((_BPW,), jnp.int32),
            pltpu.VMEM((_BPW,), jnp.int32),
            pltpu.VMEM((_H, 2 * _D), jnp.float32),
            pltpu.VMEM((_H, 2 * _D), jnp.float32),
            pltpu.VMEM((_BPW,), jnp.float32),
            pltpu.VMEM((_BPW,), jnp.float32),
            pltpu.VMEM((_L,), jnp.float32),
            pltpu.VMEM((_BPW,), jnp.float32),
            pltpu.SemaphoreType.DMA,
            pltpu.SemaphoreType.DMA,
        ],
    )

    ut2 = jnp.pad(user_table, ((0, 0), (0, 128 - _D)))
    it2 = jnp.pad(item_table, ((0, 0), (0, 128 - _D)))
    gb_b = jnp.broadcast_to(global_bias, (_L,))
    return score(user_ids.astype(jnp.int32), item_ids.astype(jnp.int32),
                 ut2, it2, user_bias, item_bias, gb_b)
